# Initial kernel scaffold; baseline (speedup 1.0000x reference)
import functools

import jax
import jax.numpy as jnp
from jax import lax
from jax.experimental import pallas as pl
from jax.experimental.pallas import tpu as pltpu
from jax.experimental.pallas import tpu_sc as plsc

VOCAB_ = 1000
EMB_ = 64
NW = 32
CHUNK = 800

QN = 1024 * 20 * 20
CN = 1024 * 50 * 20


def _gather_loop(idx_hbm, out_hbm, table_hbm, idx_v, rows_v, sem, base, nchunks):
    def step(i, carry):
        off = base + i * CHUNK
        pltpu.sync_copy(idx_hbm.at[pl.ds(off, CHUNK)], idx_v)
        pltpu.async_copy(table_hbm.at[idx_v], rows_v, sem).wait()
        pltpu.sync_copy(rows_v, out_hbm.at[pl.ds(off, CHUNK)])
        return carry

    lax.fori_loop(0, nchunks, step, 0)


def _body(q_hbm, c_hbm, table_hbm, qout_hbm, cout_hbm, idx_v, rows_v, sem):
    wid = lax.axis_index("s") * 2 + lax.axis_index("c")
    q_per_w = QN // NW
    c_per_w = CN // NW
    _gather_loop(q_hbm, qout_hbm, table_hbm, idx_v, rows_v, sem,
                 wid * q_per_w, q_per_w // CHUNK)
    _gather_loop(c_hbm, cout_hbm, table_hbm, idx_v, rows_v, sem,
                 wid * c_per_w, c_per_w // CHUNK)


@jax.jit
def _run(qidx, cidx, table):
    mesh = plsc.VectorSubcoreMesh(core_axis_name="c", subcore_axis_name="s")
    f = pl.kernel(
        _body,
        out_type=(
            jax.ShapeDtypeStruct((QN, EMB_), jnp.float32),
            jax.ShapeDtypeStruct((CN, EMB_), jnp.float32),
        ),
        mesh=mesh,
        scratch_types=[
            pltpu.VMEM((CHUNK,), jnp.int32),
            pltpu.VMEM((CHUNK, EMB_), jnp.float32),
            pltpu.SemaphoreType.DMA,
        ],
    )
    return f(qidx, cidx, table)


def kernel(question, context, char_table):
    qshape = question.shape + (EMB_,)
    cshape = context.shape + (EMB_,)
    qidx = question.reshape(-1).astype(jnp.int32)
    cidx = context.reshape(-1).astype(jnp.int32)
    q_emb, c_emb = _run(qidx, cidx, char_table)
    return (q_emb.reshape(qshape), c_emb.reshape(cshape))


# speedup vs baseline: 4.2897x; 4.2897x over previous
import functools

import jax
import jax.numpy as jnp
from jax import lax
from jax.experimental import pallas as pl
from jax.experimental.pallas import tpu as pltpu
from jax.experimental.pallas import tpu_sc as plsc

VOCAB_ = 1000
EMB_ = 64
NW = 32
CHUNK = 800

QN = 1024 * 20 * 20
CN = 1024 * 50 * 20


def _gather_loop(idx_hbm, out_hbm, table_hbm, idx_v, rows_v, sem, base, nchunks):
    def step(i, carry):
        off = base + i * CHUNK
        pltpu.sync_copy(idx_hbm.at[pl.ds(off, CHUNK)], idx_v)
        pltpu.async_copy(table_hbm.at[idx_v], rows_v, sem).wait()
        pltpu.sync_copy(rows_v, out_hbm.at[pl.ds(off, CHUNK)])
        return carry

    lax.fori_loop(0, nchunks, step, 0)


def _body(q_hbm, c_hbm, table_hbm, qout_hbm, cout_hbm, idx_v, rows_v, sem):
    wid = lax.axis_index("s") * 2 + lax.axis_index("c")
    q_per_w = QN // NW
    c_per_w = CN // NW
    _gather_loop(q_hbm, qout_hbm, table_hbm, idx_v, rows_v, sem,
                 wid * q_per_w, q_per_w // CHUNK)
    _gather_loop(c_hbm, cout_hbm, table_hbm, idx_v, rows_v, sem,
                 wid * c_per_w, c_per_w // CHUNK)


@jax.jit
def _run(qidx, cidx, table):
    mesh = plsc.VectorSubcoreMesh(core_axis_name="c", subcore_axis_name="s")
    f = pl.kernel(
        _body,
        out_type=(
            jax.ShapeDtypeStruct((QN, EMB_), jnp.float32),
            jax.ShapeDtypeStruct((CN, EMB_), jnp.float32),
        ),
        mesh=mesh,
        scratch_types=[
            pltpu.VMEM((CHUNK,), jnp.int32),
            pltpu.VMEM((CHUNK, EMB_), jnp.float32),
            pltpu.SemaphoreType.DMA,
        ],
        compiler_params=pltpu.CompilerParams(use_tc_tiling_on_sc=False),
    )
    return f(qidx, cidx, table)


def kernel(question, context, char_table):
    qshape = question.shape + (EMB_,)
    cshape = context.shape + (EMB_,)
    qidx = question.reshape(-1).astype(jnp.int32)
    cidx = context.reshape(-1).astype(jnp.int32)
    q_emb, c_emb = _run(qidx, cidx, char_table)
    return (q_emb.reshape(qshape), c_emb.reshape(cshape))
